# baseline (device time: 268559 ns/iter reference)
import jax
import jax.numpy as jnp
from jax import lax
from jax.experimental import pallas as pl
from jax.experimental.pallas import tpu as pltpu

N_DEV = 16
SQ = 256
SKV_LOCAL = 4096
HQ = 8
HKV = 2
DH = 128
D = 1024
SCALE = 0.08838834764831843

COMM_W = D + 128


def kernel(x, Wq, Wo, K_ext, V_ext):
    x2 = x.reshape(SQ, D)
    k2 = K_ext.reshape(SKV_LOCAL, HKV * DH)
    v2 = V_ext.reshape(SKV_LOCAL, HKV * DH)

    def body(x_ref, wq_ref, wo_ref, k_ref, v_ref, out_ref,
             comm_ref, send_sems, recv_sems):
        my = lax.axis_index("i")
        left = lax.rem(my + N_DEV - 1, N_DEV)
        right = lax.rem(my + 1, N_DEV)

        barrier_sem = pltpu.get_barrier_semaphore()
        for nbr in (left, right):
            pl.semaphore_signal(
                barrier_sem, inc=1,
                device_id=(nbr,), device_id_type=pl.DeviceIdType.MESH,
            )
        pl.semaphore_wait(barrier_sem, 2)

        q = jnp.dot(x_ref[...], wq_ref[...],
                    preferred_element_type=jnp.float32)

        o_acc = []
        m_acc = []
        l_acc = []
        for h in range(HQ):
            kvh = h // (HQ // HKV)
            q_h = q[:, h * DH:(h + 1) * DH]
            k_h = k_ref[:, kvh * DH:(kvh + 1) * DH]
            v_h = v_ref[:, kvh * DH:(kvh + 1) * DH]
            s = lax.dot_general(
                q_h, k_h, (((1,), (1,)), ((), ())),
                preferred_element_type=jnp.float32,
            ) * SCALE
            m_h = jnp.max(s, axis=1, keepdims=True)
            p = jnp.exp(s - m_h)
            l_h = jnp.sum(p, axis=1, keepdims=True)
            o_h = jnp.dot(p, v_h, preferred_element_type=jnp.float32)
            o_acc.append(o_h)
            m_acc.append(m_h)
            l_acc.append(l_h)

        for h in range(HQ):
            comm_ref[0, :, h * DH:(h + 1) * DH] = o_acc[h]
            comm_ref[0, :, D + h:D + h + 1] = m_acc[h]
            comm_ref[0, :, D + 8 + h:D + 8 + h + 1] = l_acc[h]

        for hop in range(N_DEV - 1):
            send_slot = hop % 2
            recv_slot = (hop + 1) % 2
            rdma = pltpu.make_async_remote_copy(
                src_ref=comm_ref.at[send_slot],
                dst_ref=comm_ref.at[recv_slot],
                send_sem=send_sems.at[send_slot],
                recv_sem=recv_sems.at[recv_slot],
                device_id=(right,),
                device_id_type=pl.DeviceIdType.MESH,
            )
            rdma.start()
            rdma.wait()

            for h in range(HQ):
                o2 = comm_ref[recv_slot, :, h * DH:(h + 1) * DH]
                m2 = comm_ref[recv_slot, :, D + h:D + h + 1]
                l2 = comm_ref[recv_slot, :, D + 8 + h:D + 8 + h + 1]
                m_new = jnp.maximum(m_acc[h], m2)
                a1 = jnp.exp(m_acc[h] - m_new)
                a2 = jnp.exp(m2 - m_new)
                o_acc[h] = o_acc[h] * a1 + o2 * a2
                l_acc[h] = l_acc[h] * a1 + l2 * a2
                m_acc[h] = m_new

        o_norm = jnp.concatenate(
            [o_acc[h] / l_acc[h] for h in range(HQ)], axis=1
        )
        out_ref[...] = jnp.dot(o_norm, wo_ref[...],
                               preferred_element_type=jnp.float32)

    out = pl.pallas_call(
        body,
        out_shape=jax.ShapeDtypeStruct((SQ, D), jnp.float32),
        in_specs=[pl.BlockSpec(memory_space=pltpu.VMEM)] * 5,
        out_specs=pl.BlockSpec(memory_space=pltpu.VMEM),
        scratch_shapes=[
            pltpu.VMEM((2, SQ, COMM_W), jnp.float32),
            pltpu.SemaphoreType.DMA((2,)),
            pltpu.SemaphoreType.DMA((2,)),
        ],
        compiler_params=pltpu.CompilerParams(collective_id=0),
    )(x2, Wq, Wo, k2, v2)
    return out.reshape(1, SQ, D)


# device time: 50190 ns/iter; 5.3508x vs baseline; 5.3508x over previous
import jax
import jax.numpy as jnp
from jax import lax
from jax.experimental import pallas as pl
from jax.experimental.pallas import tpu as pltpu

N_DEV = 16
SQ = 256
CHUNK = SQ // N_DEV
SKV_LOCAL = 4096
HQ = 8
HKV = 2
DH = 128
D = 1024
SCALE = 0.08838834764831843

COMM_W = D + 128


def kernel(x, Wq, Wo, K_ext, V_ext):
    x2 = x.reshape(SQ, D)
    k2 = K_ext.reshape(SKV_LOCAL, HKV * DH)
    v2 = V_ext.reshape(SKV_LOCAL, HKV * DH)

    def body(x_ref, wq_ref, wo_ref, k_ref, v_ref, out_ref,
             pack_ref, p1_ref, f_ref,
             s1_sem, r1_sem, s2_sem, r2_sem):
        my = lax.axis_index("i")

        barrier_sem = pltpu.get_barrier_semaphore()
        for d in range(1, N_DEV):
            peer = lax.rem(my + d, N_DEV)
            pl.semaphore_signal(
                barrier_sem, inc=1,
                device_id=(peer,), device_id_type=pl.DeviceIdType.MESH,
            )
        pl.semaphore_wait(barrier_sem, N_DEV - 1)

        q = jnp.dot(x_ref[...], wq_ref[...],
                    preferred_element_type=jnp.float32)
        for h in range(HQ):
            kvh = h // (HQ // HKV)
            q_h = q[:, h * DH:(h + 1) * DH]
            k_h = k_ref[:, kvh * DH:(kvh + 1) * DH]
            v_h = v_ref[:, kvh * DH:(kvh + 1) * DH]
            s = lax.dot_general(
                q_h, k_h, (((1,), (1,)), ((), ())),
                preferred_element_type=jnp.float32,
            ) * SCALE
            m_h = jnp.max(s, axis=1, keepdims=True)
            p = jnp.exp(s - m_h)
            l_h = jnp.sum(p, axis=1, keepdims=True)
            o_h = jnp.dot(p, v_h, preferred_element_type=jnp.float32)
            pack_ref[:, h * DH:(h + 1) * DH] = o_h
            pack_ref[:, D + h:D + h + 1] = m_h
            pack_ref[:, D + HQ + h:D + HQ + h + 1] = l_h

        rdma1 = []
        for d in range(1, N_DEV):
            dest = lax.rem(my + d, N_DEV)
            r = pltpu.make_async_remote_copy(
                src_ref=pack_ref.at[pl.ds(dest * CHUNK, CHUNK), :],
                dst_ref=p1_ref.at[d],
                send_sem=s1_sem,
                recv_sem=r1_sem,
                device_id=(dest,),
                device_id_type=pl.DeviceIdType.MESH,
            )
            r.start()
            rdma1.append(r)

        p1_ref[0] = pack_ref[pl.ds(my * CHUNK, CHUNK), :]

        for r in rdma1:
            r.wait_recv()

        arr = p1_ref[...]
        o_heads = []
        for h in range(HQ):
            m_d = arr[:, :, D + h]
            l_d = arr[:, :, D + HQ + h]
            m_mx = jnp.max(m_d, axis=0)
            w = jnp.exp(m_d - m_mx[None, :])
            l_c = jnp.sum(l_d * w, axis=0)
            o_d = arr[:, :, h * DH:(h + 1) * DH]
            o_c = jnp.sum(o_d * w[:, :, None], axis=0)
            o_heads.append(o_c / l_c[:, None])
        o_n = jnp.concatenate(o_heads, axis=1)

        final = jnp.dot(o_n, wo_ref[...],
                        preferred_element_type=jnp.float32)
        f_ref[...] = final
        out_ref[pl.ds(my * CHUNK, CHUNK), :] = final

        rdma2 = []
        for d in range(1, N_DEV):
            dest = lax.rem(my + d, N_DEV)
            r = pltpu.make_async_remote_copy(
                src_ref=f_ref,
                dst_ref=out_ref.at[pl.ds(my * CHUNK, CHUNK), :],
                send_sem=s2_sem,
                recv_sem=r2_sem,
                device_id=(dest,),
                device_id_type=pl.DeviceIdType.MESH,
            )
            r.start()
            rdma2.append(r)

        for r in rdma1:
            r.wait_send()
        for r in rdma2:
            r.wait_recv()
        for r in rdma2:
            r.wait_send()

    out = pl.pallas_call(
        body,
        out_shape=jax.ShapeDtypeStruct((SQ, D), jnp.float32),
        in_specs=[pl.BlockSpec(memory_space=pltpu.VMEM)] * 5,
        out_specs=pl.BlockSpec(memory_space=pltpu.VMEM),
        scratch_shapes=[
            pltpu.VMEM((SQ, COMM_W), jnp.float32),
            pltpu.VMEM((N_DEV, CHUNK, COMM_W), jnp.float32),
            pltpu.VMEM((CHUNK, D), jnp.float32),
            pltpu.SemaphoreType.DMA,
            pltpu.SemaphoreType.DMA,
            pltpu.SemaphoreType.DMA,
            pltpu.SemaphoreType.DMA,
        ],
        compiler_params=pltpu.CompilerParams(collective_id=0),
    )(x2, Wq, Wo, k2, v2)
    return out.reshape(1, SQ, D)


# device time: 47406 ns/iter; 5.6651x vs baseline; 1.0587x over previous
import jax
import jax.numpy as jnp
from jax import lax
from jax.experimental import pallas as pl
from jax.experimental.pallas import tpu as pltpu

N_DEV = 16
SQ = 256
CHUNK = SQ // N_DEV
HALF = SQ // 2
SKV_LOCAL = 4096
HQ = 8
HKV = 2
DH = 128
D = 1024
SCALE = 0.08838834764831843

COMM_W = D + 128


def kernel(x, Wq, Wo, K_ext, V_ext):
    x2 = x.reshape(SQ, D)
    k2 = K_ext.reshape(SKV_LOCAL, HKV * DH)
    v2 = V_ext.reshape(SKV_LOCAL, HKV * DH)

    def body(x_ref, wq_ref, wo_ref, k_ref, v_ref, out_ref,
             pack_ref, p1_ref, f_ref,
             s1_sem, r1_sem, s2_sem, r2_sem):
        my = lax.axis_index("i")

        barrier_sem = pltpu.get_barrier_semaphore()
        for d in range(1, N_DEV):
            peer = lax.rem(my + d, N_DEV)
            pl.semaphore_signal(
                barrier_sem, inc=1,
                device_id=(peer,), device_id_type=pl.DeviceIdType.MESH,
            )
        pl.semaphore_wait(barrier_sem, N_DEV - 1)

        dests = []
        rdma1 = []
        for d in range(1, N_DEV):
            dest = lax.rem(my + d, N_DEV)
            dests.append(dest)
            rdma1.append(pltpu.make_async_remote_copy(
                src_ref=pack_ref.at[pl.ds(dest * CHUNK, CHUNK), :],
                dst_ref=p1_ref.at[d],
                send_sem=s1_sem,
                recv_sem=r1_sem,
                device_id=(dest,),
                device_id_type=pl.DeviceIdType.MESH,
            ))

        qb = jnp.dot(x_ref[...].astype(jnp.bfloat16),
                     wq_ref[...].astype(jnp.bfloat16),
                     preferred_element_type=jnp.float32
                     ).astype(jnp.bfloat16)
        kb = k_ref[...].astype(jnp.bfloat16)
        vb = v_ref[...].astype(jnp.bfloat16)

        for half in range(2):
            r0 = half * HALF
            for h in range(HQ):
                kvh = h // (HQ // HKV)
                q_h = qb[r0:r0 + HALF, h * DH:(h + 1) * DH]
                k_h = kb[:, kvh * DH:(kvh + 1) * DH]
                v_h = vb[:, kvh * DH:(kvh + 1) * DH]
                s = lax.dot_general(
                    q_h, k_h, (((1,), (1,)), ((), ())),
                    preferred_element_type=jnp.float32,
                ) * SCALE
                m_h = jnp.max(s, axis=1, keepdims=True)
                p = jnp.exp(s - m_h)
                l_h = jnp.sum(p, axis=1, keepdims=True)
                o_h = jnp.dot(p.astype(jnp.bfloat16), v_h,
                              preferred_element_type=jnp.float32)
                sl = pl.ds(r0, HALF)
                pack_ref[sl, h * DH:(h + 1) * DH] = o_h.astype(jnp.bfloat16)
                pack_ref[sl, D + h:D + h + 1] = m_h.astype(jnp.bfloat16)
                pack_ref[sl, D + HQ + h:D + HQ + h + 1] = (
                    l_h.astype(jnp.bfloat16))

            for i, dest in enumerate(dests):
                row = dest * CHUNK
                in_half = jnp.logical_and(row >= r0, row < r0 + HALF)
                @pl.when(in_half)
                def _(r=rdma1[i]):
                    r.start()

        p1_ref[0] = pack_ref[pl.ds(my * CHUNK, CHUNK), :]

        for r in rdma1:
            r.wait_recv()

        arr = p1_ref[...].astype(jnp.float32)
        o_heads = []
        for h in range(HQ):
            m_d = arr[:, :, D + h]
            l_d = arr[:, :, D + HQ + h]
            m_mx = jnp.max(m_d, axis=0)
            w = jnp.exp(m_d - m_mx[None, :])
            l_c = jnp.sum(l_d * w, axis=0)
            o_d = arr[:, :, h * DH:(h + 1) * DH]
            o_c = jnp.sum(o_d * w[:, :, None], axis=0)
            o_heads.append(o_c / l_c[:, None])
        o_n = jnp.concatenate(o_heads, axis=1)

        final = jnp.dot(o_n.astype(jnp.bfloat16),
                        wo_ref[...].astype(jnp.bfloat16),
                        preferred_element_type=jnp.float32)
        f_ref[...] = final
        out_ref[pl.ds(my * CHUNK, CHUNK), :] = final

        rdma2 = []
        for d in range(1, N_DEV):
            dest = lax.rem(my + d, N_DEV)
            r = pltpu.make_async_remote_copy(
                src_ref=f_ref,
                dst_ref=out_ref.at[pl.ds(my * CHUNK, CHUNK), :],
                send_sem=s2_sem,
                recv_sem=r2_sem,
                device_id=(dest,),
                device_id_type=pl.DeviceIdType.MESH,
            )
            r.start()
            rdma2.append(r)

        for r in rdma1:
            r.wait_send()
        for r in rdma2:
            r.wait_recv()
        for r in rdma2:
            r.wait_send()

    out = pl.pallas_call(
        body,
        out_shape=jax.ShapeDtypeStruct((SQ, D), jnp.float32),
        in_specs=[pl.BlockSpec(memory_space=pltpu.VMEM)] * 5,
        out_specs=pl.BlockSpec(memory_space=pltpu.VMEM),
        scratch_shapes=[
            pltpu.VMEM((SQ, COMM_W), jnp.bfloat16),
            pltpu.VMEM((N_DEV, CHUNK, COMM_W), jnp.bfloat16),
            pltpu.VMEM((CHUNK, D), jnp.float32),
            pltpu.SemaphoreType.DMA,
            pltpu.SemaphoreType.DMA,
            pltpu.SemaphoreType.DMA,
            pltpu.SemaphoreType.DMA,
        ],
        compiler_params=pltpu.CompilerParams(collective_id=0),
    )(x2, Wq, Wo, k2, v2)
    return out.reshape(1, SQ, D)
